# Initial kernel scaffold; baseline (speedup 1.0000x reference)
import jax
import jax.numpy as jnp
from jax.experimental import pallas as pl


def kernel(h, edge_index, w):
    raise NotImplementedError("write your pallas kernel here")


# speedup vs baseline: 3.4304x; 3.4304x over previous
import functools

import jax
import jax.numpy as jnp
from jax import lax
from jax.experimental import pallas as pl
from jax.experimental.pallas import tpu as pltpu
from jax.experimental.pallas import tpu_sc as plsc

N = 10000
E = 320000
H = 8
D = 16
ROW = H * D
L = 16
NC = 2
NS = 16
NW = NC * NS
EPT = E // NW
C = 80
NCH = EPT // C
GPC = C // L


def _sqrt16(x):
    i = lax.bitcast_convert_type(x, jnp.int32)
    y = lax.bitcast_convert_type(
        jnp.int32(0x5F3759DF) - lax.shift_right_arithmetic(i, 1), jnp.float32)
    y = y * (1.5 - 0.5 * x * y * y)
    y = y * (1.5 - 0.5 * x * y * y)
    return jnp.where(x > 0.0, x * y, 0.0)


def _make_kernel():
    mesh = plsc.VectorSubcoreMesh(
        core_axis_name="c", subcore_axis_name="s", num_cores=NC,
        num_subcores=NS)

    @functools.partial(
        pl.kernel,
        out_type=(
            jax.ShapeDtypeStruct((E,), jnp.float32),
            jax.ShapeDtypeStruct((E,), jnp.float32),
        ),
        mesh=mesh,
        compiler_params=pltpu.CompilerParams(needs_layout_passes=False),
        scratch_types=[
            pltpu.VMEM((EPT,), jnp.int32),
            pltpu.VMEM((EPT,), jnp.int32),
            pltpu.VMEM((C, ROW), jnp.float32),
            pltpu.VMEM((C, ROW), jnp.float32),
            pltpu.VMEM((C,), jnp.float32),
            pltpu.VMEM((C,), jnp.float32),
            pltpu.VMEM((L,), jnp.float32),
            pltpu.VMEM((L,), jnp.float32),
            pltpu.SemaphoreType.DMA,
            pltpu.SemaphoreType.DMA,
        ],
    )
    def edge_kernel(h_hbm, si_hbm, di_hbm, w_hbm, dist_hbm, cv_hbm,
                    sidx, didx, srows, drows, distb, cvb, pv, wv,
                    sem0, sem1):
        wid = lax.axis_index("s") * NC + lax.axis_index("c")
        base = wid * EPT

        pltpu.sync_copy(si_hbm.at[pl.ds(base, EPT)], sidx)
        pltpu.sync_copy(di_hbm.at[pl.ds(base, EPT)], didx)

        pltpu.sync_copy(w_hbm, wv)
        ew = jnp.exp(wv[...])
        wv[...] = ew
        s = jnp.zeros((L,), jnp.float32)
        for h in range(H):
            s = s + plsc.load_gather(wv, [jnp.full((L,), h + 1, jnp.int32)])
        pv[...] = ew / s

        iota16 = lax.iota(jnp.int32, L)

        def chunk_body(k, carry):
            co = k * C
            cp_s = pltpu.async_copy(h_hbm.at[sidx.at[pl.ds(co, C)]],
                                    srows, sem0)
            cp_d = pltpu.async_copy(h_hbm.at[didx.at[pl.ds(co, C)]],
                                    drows, sem1)
            cp_s.wait()
            cp_d.wait()
            for g in range(GPC):
                rowi = iota16 + (g * L)
                acc = []
                for h in range(H):
                    a = jnp.zeros((L,), jnp.float32)
                    for d in range(D):
                        coli = jnp.full((L,), h * D + d, jnp.int32)
                        sv = plsc.load_gather(srows, [rowi, coli])
                        dv = plsc.load_gather(drows, [rowi, coli])
                        df = dv - sv
                        a = a + df * df
                    acc.append(a)
                wsum = jnp.zeros((L,), jnp.float32)
                s1 = jnp.zeros((L,), jnp.float32)
                s2 = jnp.zeros((L,), jnp.float32)
                for h in range(H):
                    n = _sqrt16(acc[h])
                    ph = plsc.load_gather(pv,
                                          [jnp.full((L,), h + 1, jnp.int32)])
                    wsum = wsum + n * ph
                    s1 = s1 + n
                    s2 = s2 + acc[h]
                mean = s1 * (1.0 / H)
                var = s2 * (1.0 / H) - mean * mean
                std = _sqrt16(var)
                distb[pl.ds(g * L, L)] = wsum
                cvb[pl.ds(g * L, L)] = std / (mean + 1.0)
            pltpu.sync_copy(distb, dist_hbm.at[pl.ds(base + co, C)])
            pltpu.sync_copy(cvb, cv_hbm.at[pl.ds(base + co, C)])
            return carry

        lax.fori_loop(0, NCH, chunk_body, 0)

    return edge_kernel


_edge_kernel = _make_kernel()


@jax.jit
def kernel(h, edge_index, w):
    h2 = h.reshape(N, ROW)
    si = edge_index[0]
    di = edge_index[1]
    wpad = jnp.pad(w, (1, L - H - 1), constant_values=-1e30)
    dist, cv = _edge_kernel(h2, si, di, wpad)
    return dist[:, None], cv


# speedup vs baseline: 3.5914x; 1.0469x over previous
import functools

import jax
import jax.numpy as jnp
from jax import lax
from jax.experimental import pallas as pl
from jax.experimental.pallas import tpu as pltpu
from jax.experimental.pallas import tpu_sc as plsc

N = 10000
E = 320000
H = 8
D = 16
ROW = H * D
L = 16
NC = 2
NS = 16
NW = NC * NS
EPT = E // NW
C = 80
NCH = EPT // C
GPC = C // L
NPAIR = (NCH - 1) // 2


def _sqrt16(x):
    i = lax.bitcast_convert_type(x, jnp.int32)
    y = lax.bitcast_convert_type(
        jnp.int32(0x5F3759DF) - lax.shift_right_arithmetic(i, 1), jnp.float32)
    y = y * (1.5 - 0.5 * x * y * y)
    y = y * (1.5 - 0.5 * x * y * y)
    return jnp.where(x > 0.0, x * y, 0.0)


def _make_kernel():
    mesh = plsc.VectorSubcoreMesh(
        core_axis_name="c", subcore_axis_name="s", num_cores=NC,
        num_subcores=NS)

    @functools.partial(
        pl.kernel,
        out_type=(
            jax.ShapeDtypeStruct((E,), jnp.float32),
            jax.ShapeDtypeStruct((E,), jnp.float32),
        ),
        mesh=mesh,
        compiler_params=pltpu.CompilerParams(needs_layout_passes=False),
        scratch_types=[
            pltpu.VMEM((EPT,), jnp.int32),
            pltpu.VMEM((EPT,), jnp.int32),
            pltpu.VMEM((C, ROW), jnp.float32),
            pltpu.VMEM((C, ROW), jnp.float32),
            pltpu.VMEM((C, ROW), jnp.float32),
            pltpu.VMEM((C, ROW), jnp.float32),
            pltpu.VMEM((EPT,), jnp.float32),
            pltpu.VMEM((EPT,), jnp.float32),
            pltpu.VMEM((L,), jnp.float32),
            pltpu.VMEM((L,), jnp.float32),
            pltpu.SemaphoreType.DMA,
            pltpu.SemaphoreType.DMA,
            pltpu.SemaphoreType.DMA,
            pltpu.SemaphoreType.DMA,
        ],
    )
    def edge_kernel(h_hbm, si_hbm, di_hbm, w_hbm, dist_hbm, cv_hbm,
                    sidx, didx, sra, dra, srb, drb, distb, cvb, pv, wv,
                    sa0, sa1, sb0, sb1):
        wid = lax.axis_index("s") * NC + lax.axis_index("c")
        base = wid * EPT

        pltpu.sync_copy(si_hbm.at[pl.ds(base, EPT)], sidx)
        pltpu.sync_copy(di_hbm.at[pl.ds(base, EPT)], didx)

        pltpu.sync_copy(w_hbm, wv)
        ew = jnp.exp(wv[...])
        wv[...] = ew
        s = jnp.zeros((L,), jnp.float32)
        for h in range(H):
            s = s + plsc.load_gather(wv, [jnp.full((L,), h + 1, jnp.int32)])
        pv[...] = ew / s
        probs = tuple(
            plsc.load_gather(pv, [jnp.full((L,), h + 1, jnp.int32)])
            for h in range(H))

        iota16 = lax.iota(jnp.int32, L)

        def descs(k, sr, dr, s0, s1):
            cs = pltpu.make_async_copy(h_hbm.at[sidx.at[pl.ds(k * C, C)]],
                                       sr, s0)
            cd = pltpu.make_async_copy(h_hbm.at[didx.at[pl.ds(k * C, C)]],
                                       dr, s1)
            return cs, cd

        def issue(k, sr, dr, s0, s1):
            cs, cd = descs(k, sr, dr, s0, s1)
            cs.start()
            cd.start()

        def compute(k, sr, dr, pr):
            co = k * C
            for g in range(GPC):
                rowi = iota16 + (g * L)
                acc = []
                for h in range(H):
                    a = jnp.zeros((L,), jnp.float32)
                    for d in range(D):
                        coli = jnp.full((L,), h * D + d, jnp.int32)
                        sv = plsc.load_gather(sr, [rowi, coli])
                        dv = plsc.load_gather(dr, [rowi, coli])
                        df = dv - sv
                        a = a + df * df
                    acc.append(a)
                wsum = jnp.zeros((L,), jnp.float32)
                s1 = jnp.zeros((L,), jnp.float32)
                s2 = jnp.zeros((L,), jnp.float32)
                for h in range(H):
                    n = _sqrt16(acc[h])
                    wsum = wsum + n * pr[h]
                    s1 = s1 + n
                    s2 = s2 + acc[h]
                mean = s1 * (1.0 / H)
                var = s2 * (1.0 / H) - mean * mean
                std = _sqrt16(var)
                distb[pl.ds(co + g * L, L)] = wsum
                cvb[pl.ds(co + g * L, L)] = std / (mean + 1.0)

        issue(0, sra, dra, sa0, sa1)
        issue(1, srb, drb, sb0, sb1)
        ca = descs(0, sra, dra, sa0, sa1)
        ca[0].wait()
        ca[1].wait()
        compute(0, sra, dra, probs)

        def pair_body(i, pr):
            k1 = i * 2 + 1
            issue(k1 + 1, sra, dra, sa0, sa1)
            cb = descs(k1, srb, drb, sb0, sb1)
            cb[0].wait()
            cb[1].wait()
            compute(k1, srb, drb, pr)

            @pl.when(i < NPAIR - 1)
            def _():
                issue(k1 + 2, srb, drb, sb0, sb1)

            ca2 = descs(k1 + 1, sra, dra, sa0, sa1)
            ca2[0].wait()
            ca2[1].wait()
            compute(k1 + 1, sra, dra, pr)
            return pr

        lax.fori_loop(0, NPAIR, pair_body, probs)

        pltpu.sync_copy(distb, dist_hbm.at[pl.ds(base, EPT)])
        pltpu.sync_copy(cvb, cv_hbm.at[pl.ds(base, EPT)])

    return edge_kernel


_edge_kernel = _make_kernel()


@jax.jit
def kernel(h, edge_index, w):
    h2 = h.reshape(N, ROW)
    si = edge_index[0]
    di = edge_index[1]
    wpad = jnp.pad(w, (1, L - H - 1), constant_values=-1e30)
    dist, cv = _edge_kernel(h2, si, di, wpad)
    return dist[:, None], cv


# speedup vs baseline: 6.6263x; 1.8451x over previous
import functools

import jax
import jax.numpy as jnp
from jax import lax
from jax.experimental import pallas as pl
from jax.experimental.pallas import tpu as pltpu
from jax.experimental.pallas import tpu_sc as plsc

N = 10000
E = 320000
H = 8
D = 16
ROW = H * D
ROWP = ROW + 1
L = 16
NC = 2
NS = 16
NW = NC * NS
EPT = E // NW
C = 80
NCH = EPT // C
GPC = C // L
NPAIR = (NCH - 1) // 2


def _sqrt16(x):
    i = lax.bitcast_convert_type(x, jnp.int32)
    y = lax.bitcast_convert_type(
        jnp.int32(0x5F3759DF) - lax.shift_right_arithmetic(i, 1), jnp.float32)
    y = y * (1.5 - 0.5 * x * y * y)
    y = y * (1.5 - 0.5 * x * y * y)
    return jnp.where(x > 0.0, x * y, 0.0)


def _make_kernel():
    mesh = plsc.VectorSubcoreMesh(
        core_axis_name="c", subcore_axis_name="s", num_cores=NC,
        num_subcores=NS)

    @functools.partial(
        pl.kernel,
        out_type=(
            jax.ShapeDtypeStruct((E,), jnp.float32),
            jax.ShapeDtypeStruct((E,), jnp.float32),
        ),
        mesh=mesh,
        compiler_params=pltpu.CompilerParams(needs_layout_passes=False,
                                             use_tc_tiling_on_sc=False),
        scratch_types=[
            pltpu.VMEM((EPT,), jnp.int32),
            pltpu.VMEM((EPT,), jnp.int32),
            pltpu.VMEM((C, ROWP), jnp.float32),
            pltpu.VMEM((C, ROWP), jnp.float32),
            pltpu.VMEM((C, ROWP), jnp.float32),
            pltpu.VMEM((C, ROWP), jnp.float32),
            pltpu.VMEM((EPT,), jnp.float32),
            pltpu.VMEM((EPT,), jnp.float32),
            pltpu.VMEM((L,), jnp.float32),
            pltpu.VMEM((L,), jnp.float32),
            pltpu.SemaphoreType.DMA,
            pltpu.SemaphoreType.DMA,
            pltpu.SemaphoreType.DMA,
            pltpu.SemaphoreType.DMA,
        ],
    )
    def edge_kernel(h_hbm, si_hbm, di_hbm, w_hbm, dist_hbm, cv_hbm,
                    sidx, didx, sra, dra, srb, drb, distb, cvb, pv, wv,
                    sa0, sa1, sb0, sb1):
        wid = lax.axis_index("s") * NC + lax.axis_index("c")
        base = wid * EPT

        pltpu.sync_copy(si_hbm.at[pl.ds(base, EPT)], sidx)
        pltpu.sync_copy(di_hbm.at[pl.ds(base, EPT)], didx)

        pltpu.sync_copy(w_hbm, wv)
        ew = jnp.exp(wv[...])
        wv[...] = ew
        s = jnp.zeros((L,), jnp.float32)
        for h in range(H):
            s = s + plsc.load_gather(wv, [jnp.full((L,), h + 1, jnp.int32)])
        pv[...] = ew / s

        iota16 = lax.iota(jnp.int32, L)

        def descs(k, sr, dr, s0, s1):
            cs = pltpu.make_async_copy(h_hbm.at[sidx.at[pl.ds(k * C, C)]],
                                       sr, s0)
            cd = pltpu.make_async_copy(h_hbm.at[didx.at[pl.ds(k * C, C)]],
                                       dr, s1)
            return cs, cd

        def issue(k, sr, dr, s0, s1):
            cs, cd = descs(k, sr, dr, s0, s1)
            cs.start()
            cd.start()

        def compute(k, sr, dr):
            co = k * C
            for g in range(GPC):
                rowi = iota16 + (g * L)
                wsum = jnp.zeros((L,), jnp.float32)
                s1 = jnp.zeros((L,), jnp.float32)
                s2 = jnp.zeros((L,), jnp.float32)
                for half in range(2):
                    accs = [jnp.zeros((L,), jnp.float32) for _ in range(4)]
                    for d in range(D):
                        for j in range(4):
                            coli = jnp.full((L,), (half * 4 + j) * D + d,
                                            jnp.int32)
                            sv = plsc.load_gather(sr, [rowi, coli])
                            dv = plsc.load_gather(dr, [rowi, coli])
                            df = dv - sv
                            accs[j] = accs[j] + df * df
                    for j in range(4):
                        h = half * 4 + j
                        a = accs[j]
                        n = _sqrt16(a)
                        ph = plsc.load_gather(
                            pv, [jnp.full((L,), h + 1, jnp.int32)])
                        wsum = wsum + n * ph
                        s1 = s1 + n
                        s2 = s2 + a
                mean = s1 * (1.0 / H)
                var = s2 * (1.0 / H) - mean * mean
                std = _sqrt16(var)
                distb[pl.ds(co + g * L, L)] = wsum
                cvb[pl.ds(co + g * L, L)] = std / (mean + 1.0)

        issue(0, sra, dra, sa0, sa1)
        issue(1, srb, drb, sb0, sb1)
        ca = descs(0, sra, dra, sa0, sa1)
        ca[0].wait()
        ca[1].wait()
        compute(0, sra, dra)

        def pair_body(i, carry):
            k1 = i * 2 + 1
            issue(k1 + 1, sra, dra, sa0, sa1)
            cb = descs(k1, srb, drb, sb0, sb1)
            cb[0].wait()
            cb[1].wait()
            compute(k1, srb, drb)

            @pl.when(i < NPAIR - 1)
            def _():
                issue(k1 + 2, srb, drb, sb0, sb1)

            ca2 = descs(k1 + 1, sra, dra, sa0, sa1)
            ca2[0].wait()
            ca2[1].wait()
            compute(k1 + 1, sra, dra)
            return carry

        lax.fori_loop(0, NPAIR, pair_body, 0)

        pltpu.sync_copy(distb, dist_hbm.at[pl.ds(base, EPT)])
        pltpu.sync_copy(cvb, cv_hbm.at[pl.ds(base, EPT)])

    return edge_kernel


_edge_kernel = _make_kernel()


@jax.jit
def kernel(h, edge_index, w):
    h2 = jnp.pad(h.reshape(N, ROW), ((0, 0), (0, ROWP - ROW)))
    si = edge_index[0]
    di = edge_index[1]
    wpad = jnp.pad(w, (1, L - H - 1), constant_values=-1e30)
    dist, cv = _edge_kernel(h2, si, di, wpad)
    return dist[:, None], cv


# speedup vs baseline: 20.9628x; 3.1636x over previous
import functools

import jax
import jax.numpy as jnp
from jax import lax
from jax.experimental import pallas as pl
from jax.experimental.pallas import tpu as pltpu
from jax.experimental.pallas import tpu_sc as plsc

N = 10000
E = 320000
H = 8
D = 16
ROW = H * D
ROWP = ROW
L = 16
NC = 2
NS = 16
NW = NC * NS
EPT = E // NW
C = 80
NCH = EPT // C
GPC = C // L
NPAIR = (NCH - 1) // 2


def _sqrt16(x):
    i = lax.bitcast_convert_type(x, jnp.int32)
    y = lax.bitcast_convert_type(
        jnp.int32(0x5F3759DF) - lax.shift_right_arithmetic(i, 1), jnp.float32)
    y = y * (1.5 - 0.5 * x * y * y)
    y = y * (1.5 - 0.5 * x * y * y)
    return jnp.where(x > 0.0, x * y, 0.0)


def _make_kernel():
    mesh = plsc.VectorSubcoreMesh(
        core_axis_name="c", subcore_axis_name="s", num_cores=NC,
        num_subcores=NS)

    @functools.partial(
        pl.kernel,
        out_type=(
            jax.ShapeDtypeStruct((E,), jnp.float32),
            jax.ShapeDtypeStruct((E,), jnp.float32),
        ),
        mesh=mesh,
        compiler_params=pltpu.CompilerParams(needs_layout_passes=False,
                                             use_tc_tiling_on_sc=False),
        scratch_types=[
            pltpu.VMEM((EPT,), jnp.int32),
            pltpu.VMEM((EPT,), jnp.int32),
            pltpu.VMEM((C, ROWP), jnp.float32),
            pltpu.VMEM((C, ROWP), jnp.float32),
            pltpu.VMEM((C, ROWP), jnp.float32),
            pltpu.VMEM((C, ROWP), jnp.float32),
            pltpu.VMEM((EPT,), jnp.float32),
            pltpu.VMEM((EPT,), jnp.float32),
            pltpu.VMEM((L,), jnp.float32),
            pltpu.VMEM((L,), jnp.float32),
            pltpu.VMEM((ROW * L,), jnp.int32),
            pltpu.SemaphoreType.DMA,
            pltpu.SemaphoreType.DMA,
            pltpu.SemaphoreType.DMA,
            pltpu.SemaphoreType.DMA,
        ],
    )
    def edge_kernel(h_hbm, si_hbm, di_hbm, w_hbm, dist_hbm, cv_hbm,
                    sidx, didx, sra, dra, srb, drb, distb, cvb, pv, wv,
                    ctab, sa0, sa1, sb0, sb1):
        wid = lax.axis_index("s") * NC + lax.axis_index("c")
        base = wid * EPT

        pltpu.sync_copy(si_hbm.at[pl.ds(base, EPT)], sidx)
        pltpu.sync_copy(di_hbm.at[pl.ds(base, EPT)], didx)

        pltpu.sync_copy(w_hbm, wv)
        ew = jnp.exp(wv[...])
        wv[...] = ew
        s = jnp.zeros((L,), jnp.float32)
        for h in range(H):
            s = s + plsc.load_gather(wv, [jnp.full((L,), h + 1, jnp.int32)])
        pv[...] = ew / s

        iota16 = lax.iota(jnp.int32, L)

        def fill_body(t, carry):
            hh = t // D
            d = t - hh * D
            rot = jnp.bitwise_and(iota16 + d, 15) + (hh * D)
            ctab[pl.ds(t * L, L)] = rot
            return carry

        lax.fori_loop(0, H * D, fill_body, 0)

        def descs(k, sr, dr, s0, s1):
            cs = pltpu.make_async_copy(h_hbm.at[sidx.at[pl.ds(k * C, C)]],
                                       sr, s0)
            cd = pltpu.make_async_copy(h_hbm.at[didx.at[pl.ds(k * C, C)]],
                                       dr, s1)
            return cs, cd

        def issue(k, sr, dr, s0, s1):
            cs, cd = descs(k, sr, dr, s0, s1)
            cs.start()
            cd.start()

        def compute(k, sr, dr):
            co = k * C

            def group_body(g, carry):
                rowi = iota16 + g * L
                wsum = jnp.zeros((L,), jnp.float32)
                s1 = jnp.zeros((L,), jnp.float32)
                s2 = jnp.zeros((L,), jnp.float32)
                for half in range(2):
                    accs = [jnp.zeros((L,), jnp.float32) for _ in range(4)]
                    for d in range(D):
                        for j in range(4):
                            hh = half * 4 + j
                            coli = ctab[pl.ds((hh * D + d) * L, L)]
                            sv = plsc.load_gather(sr, [rowi, coli])
                            dv = plsc.load_gather(dr, [rowi, coli])
                            df = dv - sv
                            accs[j] = accs[j] + df * df
                    for j in range(4):
                        h = half * 4 + j
                        a = accs[j]
                        n = _sqrt16(a)
                        ph = plsc.load_gather(
                            pv, [jnp.full((L,), h + 1, jnp.int32)])
                        wsum = wsum + n * ph
                        s1 = s1 + n
                        s2 = s2 + a
                mean = s1 * (1.0 / H)
                var = s2 * (1.0 / H) - mean * mean
                std = _sqrt16(var)
                distb[pl.ds(co + g * L, L)] = wsum
                cvb[pl.ds(co + g * L, L)] = std / (mean + 1.0)
                return carry

            lax.fori_loop(0, GPC, group_body, 0)

        issue(0, sra, dra, sa0, sa1)
        issue(1, srb, drb, sb0, sb1)
        ca = descs(0, sra, dra, sa0, sa1)
        ca[0].wait()
        ca[1].wait()
        compute(0, sra, dra)

        def pair_body(i, carry):
            k1 = i * 2 + 1
            issue(k1 + 1, sra, dra, sa0, sa1)
            cb = descs(k1, srb, drb, sb0, sb1)
            cb[0].wait()
            cb[1].wait()
            compute(k1, srb, drb)

            @pl.when(i < NPAIR - 1)
            def _():
                issue(k1 + 2, srb, drb, sb0, sb1)

            ca2 = descs(k1 + 1, sra, dra, sa0, sa1)
            ca2[0].wait()
            ca2[1].wait()
            compute(k1 + 1, sra, dra)
            return carry

        lax.fori_loop(0, NPAIR, pair_body, 0)

        pltpu.sync_copy(distb, dist_hbm.at[pl.ds(base, EPT)])
        pltpu.sync_copy(cvb, cv_hbm.at[pl.ds(base, EPT)])

    return edge_kernel


_edge_kernel = _make_kernel()


@jax.jit
def kernel(h, edge_index, w):
    h2 = h.reshape(N, ROW)
    si = edge_index[0]
    di = edge_index[1]
    wpad = jnp.pad(w, (1, L - H - 1), constant_values=-1e30)
    dist, cv = _edge_kernel(h2, si, di, wpad)
    return dist[:, None], cv


# speedup vs baseline: 25.8735x; 1.2343x over previous
import functools

import jax
import jax.numpy as jnp
from jax import lax
from jax.experimental import pallas as pl
from jax.experimental.pallas import tpu as pltpu
from jax.experimental.pallas import tpu_sc as plsc

N = 10000
E = 320000
H = 8
D = 16
ROW = H * D
CW = ROW // 2
HP = H // 2
L = 16
NC = 2
NS = 16
NW = NC * NS
EPT = E // NW
C = 80
NCH = EPT // C
GPC = C // L
NPAIR = (NCH - 1) // 2


def _sqrt16(x):
    i = lax.bitcast_convert_type(x, jnp.int32)
    y = lax.bitcast_convert_type(
        jnp.int32(0x5F3759DF) - lax.shift_right_arithmetic(i, 1), jnp.float32)
    y = y * (1.5 - 0.5 * x * y * y)
    y = y * (1.5 - 0.5 * x * y * y)
    return jnp.where(x > 0.0, x * y, 0.0)


def _make_kernel():
    mesh = plsc.VectorSubcoreMesh(
        core_axis_name="c", subcore_axis_name="s", num_cores=NC,
        num_subcores=NS)

    @functools.partial(
        pl.kernel,
        out_type=(
            jax.ShapeDtypeStruct((E,), jnp.float32),
            jax.ShapeDtypeStruct((E,), jnp.float32),
        ),
        mesh=mesh,
        compiler_params=pltpu.CompilerParams(needs_layout_passes=False,
                                             use_tc_tiling_on_sc=False),
        scratch_types=[
            pltpu.VMEM((EPT,), jnp.int32),
            pltpu.VMEM((EPT,), jnp.int32),
            pltpu.VMEM((C, CW), jnp.int32),
            pltpu.VMEM((C, CW), jnp.int32),
            pltpu.VMEM((C, CW), jnp.int32),
            pltpu.VMEM((C, CW), jnp.int32),
            pltpu.VMEM((EPT,), jnp.float32),
            pltpu.VMEM((EPT,), jnp.float32),
            pltpu.VMEM((L,), jnp.float32),
            pltpu.VMEM((L,), jnp.float32),
            pltpu.VMEM((HP * D * L,), jnp.int32),
            pltpu.SemaphoreType.DMA,
            pltpu.SemaphoreType.DMA,
            pltpu.SemaphoreType.DMA,
            pltpu.SemaphoreType.DMA,
        ],
    )
    def edge_kernel(h_hbm, si_hbm, di_hbm, w_hbm, dist_hbm, cv_hbm,
                    sidx, didx, sra, dra, srb, drb, distb, cvb, pv, wv,
                    ctab, sa0, sa1, sb0, sb1):
        wid = lax.axis_index("s") * NC + lax.axis_index("c")
        base = wid * EPT

        pltpu.sync_copy(si_hbm.at[pl.ds(base, EPT)], sidx)
        pltpu.sync_copy(di_hbm.at[pl.ds(base, EPT)], didx)

        pltpu.sync_copy(w_hbm, wv)
        ew = jnp.exp(wv[...])
        wv[...] = ew
        s = jnp.zeros((L,), jnp.float32)
        for h in range(H):
            s = s + plsc.load_gather(wv, [jnp.full((L,), h + 1, jnp.int32)])
        pv[...] = ew / s

        iota16 = lax.iota(jnp.int32, L)

        def fill_body(t, carry):
            hp = t // D
            d = t - hp * D
            rot = jnp.bitwise_and(iota16 + d, 15) + (hp * D)
            ctab[pl.ds(t * L, L)] = rot
            return carry

        lax.fori_loop(0, HP * D, fill_body, 0)

        def descs(k, sr, dr, s0, s1):
            cs = pltpu.make_async_copy(h_hbm.at[sidx.at[pl.ds(k * C, C)]],
                                       sr, s0)
            cd = pltpu.make_async_copy(h_hbm.at[didx.at[pl.ds(k * C, C)]],
                                       dr, s1)
            return cs, cd

        def issue(k, sr, dr, s0, s1):
            cs, cd = descs(k, sr, dr, s0, s1)
            cs.start()
            cd.start()

        def compute(k, sr, dr):
            co = k * C

            def group_body(g, carry):
                rowi = iota16 + g * L
                wsum = jnp.zeros((L,), jnp.float32)
                s1 = jnp.zeros((L,), jnp.float32)
                s2 = jnp.zeros((L,), jnp.float32)
                for hp in range(HP):
                    alo = jnp.zeros((L,), jnp.float32)
                    ahi = jnp.zeros((L,), jnp.float32)
                    for d in range(D):
                        coli = ctab[pl.ds((hp * D + d) * L, L)]
                        sv = plsc.load_gather(sr, [rowi, coli])
                        dv = plsc.load_gather(dr, [rowi, coli])
                        sl, sh = plsc.unpack(
                            plsc.bitcast(sv, jnp.bfloat16),
                            format=plsc.PackFormat.INTERLEAVED)
                        dl, dh = plsc.unpack(
                            plsc.bitcast(dv, jnp.bfloat16),
                            format=plsc.PackFormat.INTERLEAVED)
                        dfl = dl - sl
                        alo = alo + dfl * dfl
                        dfh = dh - sh
                        ahi = ahi + dfh * dfh
                    for a, h in ((alo, hp), (ahi, hp + 4)):
                        n = _sqrt16(a)
                        ph = plsc.load_gather(
                            pv, [jnp.full((L,), h + 1, jnp.int32)])
                        wsum = wsum + n * ph
                        s1 = s1 + n
                        s2 = s2 + a
                mean = s1 * (1.0 / H)
                var = s2 * (1.0 / H) - mean * mean
                std = _sqrt16(var)
                distb[pl.ds(co + g * L, L)] = wsum
                cvb[pl.ds(co + g * L, L)] = std / (mean + 1.0)
                return carry

            lax.fori_loop(0, GPC, group_body, 0)

        issue(0, sra, dra, sa0, sa1)
        issue(1, srb, drb, sb0, sb1)
        ca = descs(0, sra, dra, sa0, sa1)
        ca[0].wait()
        ca[1].wait()
        compute(0, sra, dra)

        def pair_body(i, carry):
            k1 = i * 2 + 1
            issue(k1 + 1, sra, dra, sa0, sa1)
            cb = descs(k1, srb, drb, sb0, sb1)
            cb[0].wait()
            cb[1].wait()
            compute(k1, srb, drb)

            @pl.when(i < NPAIR - 1)
            def _():
                issue(k1 + 2, srb, drb, sb0, sb1)

            ca2 = descs(k1 + 1, sra, dra, sa0, sa1)
            ca2[0].wait()
            ca2[1].wait()
            compute(k1 + 1, sra, dra)
            return carry

        lax.fori_loop(0, NPAIR, pair_body, 0)

        pltpu.sync_copy(distb, dist_hbm.at[pl.ds(base, EPT)])
        pltpu.sync_copy(cvb, cv_hbm.at[pl.ds(base, EPT)])

    return edge_kernel


_edge_kernel = _make_kernel()


@jax.jit
def kernel(h, edge_index, w):
    hb = h.astype(jnp.bfloat16)
    hpair = jnp.stack([hb[:, :HP, :], hb[:, HP:, :]], axis=-1)
    h2 = lax.bitcast_convert_type(hpair, jnp.int32).reshape(N, CW)
    si = edge_index[0]
    di = edge_index[1]
    wpad = jnp.pad(w, (1, L - H - 1), constant_values=-1e30)
    dist, cv = _edge_kernel(h2, si, di, wpad)
    return dist[:, None], cv


# speedup vs baseline: 28.7833x; 1.1125x over previous
import functools

import jax
import jax.numpy as jnp
from jax import lax
from jax.experimental import pallas as pl
from jax.experimental.pallas import tpu as pltpu
from jax.experimental.pallas import tpu_sc as plsc

N = 10000
E = 320000
H = 8
D = 16
ROW = H * D
CW = ROW // 2
HP = H // 2
L = 16
NC = 2
NS = 16
NW = NC * NS
EPT = E // NW
C = 80
NCH = EPT // C
GPC = C // L
NPAIR = (NCH - 1) // 2


def _sqrt16(x):
    i = lax.bitcast_convert_type(x, jnp.int32)
    y = lax.bitcast_convert_type(
        jnp.int32(0x5F3759DF) - lax.shift_right_arithmetic(i, 1), jnp.float32)
    y = y * (1.5 - 0.5 * x * y * y)
    y = y * (1.5 - 0.5 * x * y * y)
    return jnp.where(x > 0.0, x * y, 0.0)


def _make_kernel():
    mesh = plsc.VectorSubcoreMesh(
        core_axis_name="c", subcore_axis_name="s", num_cores=NC,
        num_subcores=NS)

    @functools.partial(
        pl.kernel,
        out_type=(
            jax.ShapeDtypeStruct((E,), jnp.float32),
            jax.ShapeDtypeStruct((E,), jnp.float32),
        ),
        mesh=mesh,
        compiler_params=pltpu.CompilerParams(needs_layout_passes=False,
                                             use_tc_tiling_on_sc=False),
        scratch_types=[
            pltpu.VMEM((EPT,), jnp.int32),
            pltpu.VMEM((EPT,), jnp.int32),
            pltpu.VMEM((C, CW), jnp.int32),
            pltpu.VMEM((C, CW), jnp.int32),
            pltpu.VMEM((C, CW), jnp.int32),
            pltpu.VMEM((C, CW), jnp.int32),
            pltpu.VMEM((EPT,), jnp.float32),
            pltpu.VMEM((EPT,), jnp.float32),
            pltpu.VMEM((L,), jnp.float32),
            pltpu.VMEM((L,), jnp.float32),
            pltpu.VMEM((HP * D * L,), jnp.int32),
            pltpu.SemaphoreType.DMA,
            pltpu.SemaphoreType.DMA,
            pltpu.SemaphoreType.DMA,
            pltpu.SemaphoreType.DMA,
        ],
    )
    def edge_kernel(h_hbm, si_hbm, di_hbm, w_hbm, dist_hbm, cv_hbm,
                    sidx, didx, sra, dra, srb, drb, distb, cvb, pv, wv,
                    ctab, sa0, sa1, sb0, sb1):
        wid = lax.axis_index("s") * NC + lax.axis_index("c")
        base = wid * EPT

        pltpu.sync_copy(si_hbm.at[pl.ds(base, EPT)], sidx)
        pltpu.sync_copy(di_hbm.at[pl.ds(base, EPT)], didx)

        pltpu.sync_copy(w_hbm, wv)
        ew = jnp.exp(wv[...])
        wv[...] = ew
        s = jnp.zeros((L,), jnp.float32)
        for h in range(H):
            s = s + plsc.load_gather(wv, [jnp.full((L,), h + 1, jnp.int32)])
        pv[...] = ew / s

        iota16 = lax.iota(jnp.int32, L)

        def fill_body(t, carry):
            hp = t // D
            d = t - hp * D
            rot = jnp.bitwise_and(iota16 + d, 15) + (hp * D)
            ctab[pl.ds(t * L, L)] = rot
            return carry

        lax.fori_loop(0, HP * D, fill_body, 0)

        def descs(k, sr, dr, s0, s1):
            cs = pltpu.make_async_copy(h_hbm.at[sidx.at[pl.ds(k * C, C)]],
                                       sr, s0)
            cd = pltpu.make_async_copy(h_hbm.at[didx.at[pl.ds(k * C, C)]],
                                       dr, s1)
            return cs, cd

        def issue(k, sr, dr, s0, s1):
            cs, cd = descs(k, sr, dr, s0, s1)
            cs.start()
            cd.start()

        def compute(k, sr, dr):
            co = k * C

            def group_body(g, carry):
                rowi = iota16 + g * L
                wsum = jnp.zeros((L,), jnp.float32)
                s1 = jnp.zeros((L,), jnp.float32)
                s2 = jnp.zeros((L,), jnp.float32)
                for hp in range(HP):
                    alo = jnp.zeros((L,), jnp.float32)
                    ahi = jnp.zeros((L,), jnp.float32)
                    for d in range(D):
                        coli = ctab[pl.ds((hp * D + d) * L, L)]
                        sv = plsc.load_gather(sr, [rowi, coli])
                        dv = plsc.load_gather(dr, [rowi, coli])
                        df2 = (plsc.bitcast(dv, jnp.bfloat16)
                               - plsc.bitcast(sv, jnp.bfloat16))
                        dfl, dfh = plsc.unpack(
                            df2, format=plsc.PackFormat.INTERLEAVED)
                        alo = alo + dfl * dfl
                        ahi = ahi + dfh * dfh
                    for a, h in ((alo, hp), (ahi, hp + 4)):
                        n = _sqrt16(a)
                        ph = plsc.load_gather(
                            pv, [jnp.full((L,), h + 1, jnp.int32)])
                        wsum = wsum + n * ph
                        s1 = s1 + n
                        s2 = s2 + a
                mean = s1 * (1.0 / H)
                var = s2 * (1.0 / H) - mean * mean
                std = _sqrt16(var)
                distb[pl.ds(co + g * L, L)] = wsum
                cvb[pl.ds(co + g * L, L)] = std / (mean + 1.0)
                return carry

            lax.fori_loop(0, GPC, group_body, 0)

        issue(0, sra, dra, sa0, sa1)
        issue(1, srb, drb, sb0, sb1)
        ca = descs(0, sra, dra, sa0, sa1)
        ca[0].wait()
        ca[1].wait()
        compute(0, sra, dra)

        def pair_body(i, carry):
            k1 = i * 2 + 1
            issue(k1 + 1, sra, dra, sa0, sa1)
            cb = descs(k1, srb, drb, sb0, sb1)
            cb[0].wait()
            cb[1].wait()
            compute(k1, srb, drb)

            @pl.when(i < NPAIR - 1)
            def _():
                issue(k1 + 2, srb, drb, sb0, sb1)

            ca2 = descs(k1 + 1, sra, dra, sa0, sa1)
            ca2[0].wait()
            ca2[1].wait()
            compute(k1 + 1, sra, dra)
            return carry

        lax.fori_loop(0, NPAIR, pair_body, 0)

        pltpu.sync_copy(distb, dist_hbm.at[pl.ds(base, EPT)])
        pltpu.sync_copy(cvb, cv_hbm.at[pl.ds(base, EPT)])

    return edge_kernel


_edge_kernel = _make_kernel()


@jax.jit
def kernel(h, edge_index, w):
    hb = h.astype(jnp.bfloat16)
    hpair = jnp.stack([hb[:, :HP, :], hb[:, HP:, :]], axis=-1)
    h2 = lax.bitcast_convert_type(hpair, jnp.int32).reshape(N, CW)
    si = edge_index[0]
    di = edge_index[1]
    wpad = jnp.pad(w, (1, L - H - 1), constant_values=-1e30)
    dist, cv = _edge_kernel(h2, si, di, wpad)
    return dist[:, None], cv


# speedup vs baseline: 29.2333x; 1.0156x over previous
import functools

import jax
import jax.numpy as jnp
from jax import lax
from jax.experimental import pallas as pl
from jax.experimental.pallas import tpu as pltpu
from jax.experimental.pallas import tpu_sc as plsc

N = 10000
E = 320000
H = 8
D = 16
ROW = H * D
CW = ROW // 2
HP = H // 2
L = 16
NC = 2
NS = 16
NW = NC * NS
EPT = E // NW
C = 80
NCH = EPT // C
GPC = C // L
NPAIR = (NCH - 1) // 2


def _sqrt16(x):
    i = lax.bitcast_convert_type(x, jnp.int32)
    y = lax.bitcast_convert_type(
        jnp.int32(0x5F3759DF) - lax.shift_right_arithmetic(i, 1), jnp.float32)
    y = y * (1.5 - 0.5 * x * y * y)
    y = y * (1.5 - 0.5 * x * y * y)
    return jnp.where(x > 0.0, x * y, 0.0)


def _make_kernel():
    mesh = plsc.VectorSubcoreMesh(
        core_axis_name="c", subcore_axis_name="s", num_cores=NC,
        num_subcores=NS)

    @functools.partial(
        pl.kernel,
        out_type=(
            jax.ShapeDtypeStruct((E,), jnp.float32),
            jax.ShapeDtypeStruct((E,), jnp.float32),
        ),
        mesh=mesh,
        compiler_params=pltpu.CompilerParams(needs_layout_passes=False,
                                             use_tc_tiling_on_sc=False),
        scratch_types=[
            pltpu.VMEM((EPT,), jnp.int32),
            pltpu.VMEM((EPT,), jnp.int32),
            pltpu.VMEM((C, CW), jnp.int32),
            pltpu.VMEM((C, CW), jnp.int32),
            pltpu.VMEM((C, CW), jnp.int32),
            pltpu.VMEM((C, CW), jnp.int32),
            pltpu.VMEM((EPT,), jnp.float32),
            pltpu.VMEM((EPT,), jnp.float32),
            pltpu.VMEM((L,), jnp.float32),
            pltpu.VMEM((L,), jnp.float32),
            pltpu.VMEM((HP * D * L,), jnp.int32),
            pltpu.SemaphoreType.DMA,
            pltpu.SemaphoreType.DMA,
            pltpu.SemaphoreType.DMA,
            pltpu.SemaphoreType.DMA,
        ],
    )
    def edge_kernel(h_hbm, si_hbm, di_hbm, w_hbm, dist_hbm, cv_hbm,
                    sidx, didx, sra, dra, srb, drb, distb, cvb, pv, wv,
                    ctab, sa0, sa1, sb0, sb1):
        wid = lax.axis_index("s") * NC + lax.axis_index("c")
        base = wid * EPT

        pltpu.sync_copy(si_hbm.at[pl.ds(base, EPT)], sidx)
        pltpu.sync_copy(di_hbm.at[pl.ds(base, EPT)], didx)

        pltpu.sync_copy(w_hbm, wv)
        ew = jnp.exp(wv[...])
        wv[...] = ew
        s = jnp.zeros((L,), jnp.float32)
        for h in range(H):
            s = s + plsc.load_gather(wv, [jnp.full((L,), h + 1, jnp.int32)])
        pv[...] = ew / s

        iota16 = lax.iota(jnp.int32, L)

        def fill_body(t, carry):
            hp = t // D
            d = t - hp * D
            rot = jnp.bitwise_and(iota16 + d, 15) + (hp * D)
            ctab[pl.ds(t * L, L)] = rot
            return carry

        lax.fori_loop(0, HP * D, fill_body, 0)

        def descs(k, sr, dr, s0, s1):
            cs = pltpu.make_async_copy(h_hbm.at[sidx.at[pl.ds(k * C, C)]],
                                       sr, s0)
            cd = pltpu.make_async_copy(h_hbm.at[didx.at[pl.ds(k * C, C)]],
                                       dr, s1)
            return cs, cd

        def issue(k, sr, dr, s0, s1):
            cs, cd = descs(k, sr, dr, s0, s1)
            cs.start()
            cd.start()

        def compute(k, sr, dr):
            co = k * C

            def group_body(g, carry):
                rowi = iota16 + g * L
                wsum = jnp.zeros((L,), jnp.float32)
                s1 = jnp.zeros((L,), jnp.float32)
                s2 = jnp.zeros((L,), jnp.float32)
                for hp in range(HP):
                    alo = jnp.zeros((L,), jnp.float32)
                    ahi = jnp.zeros((L,), jnp.float32)
                    for d in range(D):
                        coli = ctab[pl.ds((hp * D + d) * L, L)]
                        sv = plsc.load_gather(sr, [rowi, coli])
                        dv = plsc.load_gather(dr, [rowi, coli])
                        df2 = (plsc.bitcast(dv, jnp.bfloat16)
                               - plsc.bitcast(sv, jnp.bfloat16))
                        sq2 = df2 * df2
                        sql, sqh = plsc.unpack(
                            sq2, format=plsc.PackFormat.INTERLEAVED)
                        alo = alo + sql
                        ahi = ahi + sqh
                    for a, h in ((alo, hp), (ahi, hp + 4)):
                        n = _sqrt16(a)
                        ph = plsc.load_gather(
                            pv, [jnp.full((L,), h + 1, jnp.int32)])
                        wsum = wsum + n * ph
                        s1 = s1 + n
                        s2 = s2 + a
                mean = s1 * (1.0 / H)
                var = s2 * (1.0 / H) - mean * mean
                std = _sqrt16(var)
                distb[pl.ds(co + g * L, L)] = wsum
                cvb[pl.ds(co + g * L, L)] = std / (mean + 1.0)
                return carry

            lax.fori_loop(0, GPC, group_body, 0)

        issue(0, sra, dra, sa0, sa1)
        issue(1, srb, drb, sb0, sb1)
        ca = descs(0, sra, dra, sa0, sa1)
        ca[0].wait()
        ca[1].wait()
        compute(0, sra, dra)

        def pair_body(i, carry):
            k1 = i * 2 + 1
            issue(k1 + 1, sra, dra, sa0, sa1)
            cb = descs(k1, srb, drb, sb0, sb1)
            cb[0].wait()
            cb[1].wait()
            compute(k1, srb, drb)

            @pl.when(i < NPAIR - 1)
            def _():
                issue(k1 + 2, srb, drb, sb0, sb1)

            ca2 = descs(k1 + 1, sra, dra, sa0, sa1)
            ca2[0].wait()
            ca2[1].wait()
            compute(k1 + 1, sra, dra)
            return carry

        lax.fori_loop(0, NPAIR, pair_body, 0)

        pltpu.sync_copy(distb, dist_hbm.at[pl.ds(base, EPT)])
        pltpu.sync_copy(cvb, cv_hbm.at[pl.ds(base, EPT)])

    return edge_kernel


_edge_kernel = _make_kernel()


@jax.jit
def kernel(h, edge_index, w):
    hb = h.astype(jnp.bfloat16)
    hpair = jnp.stack([hb[:, :HP, :], hb[:, HP:, :]], axis=-1)
    h2 = lax.bitcast_convert_type(hpair, jnp.int32).reshape(N, CW)
    si = edge_index[0]
    di = edge_index[1]
    wpad = jnp.pad(w, (1, L - H - 1), constant_values=-1e30)
    dist, cv = _edge_kernel(h2, si, di, wpad)
    return dist[:, None], cv


# speedup vs baseline: 30.1957x; 1.0329x over previous
import functools

import jax
import jax.numpy as jnp
from jax import lax
from jax.experimental import pallas as pl
from jax.experimental.pallas import tpu as pltpu
from jax.experimental.pallas import tpu_sc as plsc

N = 10000
E = 320000
H = 8
D = 16
ROW = H * D
CW = ROW // 2
HP = H // 2
L = 16
NC = 2
NS = 16
NW = NC * NS
EPT = E // NW
C = 80
NCH = EPT // C
GPC = C // L
NTRI = (NCH - 2) // 3


def _sqrt16(x):
    i = lax.bitcast_convert_type(x, jnp.int32)
    y = lax.bitcast_convert_type(
        jnp.int32(0x5F3759DF) - lax.shift_right_arithmetic(i, 1), jnp.float32)
    y = y * (1.5 - 0.5 * x * y * y)
    y = y * (1.5 - 0.5 * x * y * y)
    return jnp.where(x > 0.0, x * y, 0.0)


def _make_kernel():
    mesh = plsc.VectorSubcoreMesh(
        core_axis_name="c", subcore_axis_name="s", num_cores=NC,
        num_subcores=NS)

    @functools.partial(
        pl.kernel,
        out_type=(
            jax.ShapeDtypeStruct((E,), jnp.float32),
            jax.ShapeDtypeStruct((E,), jnp.float32),
        ),
        mesh=mesh,
        compiler_params=pltpu.CompilerParams(needs_layout_passes=False,
                                             use_tc_tiling_on_sc=False),
        scratch_types=[
            pltpu.VMEM((EPT,), jnp.int32),
            pltpu.VMEM((EPT,), jnp.int32),
            pltpu.VMEM((C, CW), jnp.int32),
            pltpu.VMEM((C, CW), jnp.int32),
            pltpu.VMEM((C, CW), jnp.int32),
            pltpu.VMEM((C, CW), jnp.int32),
            pltpu.VMEM((C, CW), jnp.int32),
            pltpu.VMEM((C, CW), jnp.int32),
            pltpu.VMEM((EPT,), jnp.float32),
            pltpu.VMEM((EPT,), jnp.float32),
            pltpu.VMEM((L,), jnp.float32),
            pltpu.VMEM((L,), jnp.float32),
            pltpu.VMEM((HP * D * L,), jnp.int32),
            pltpu.SemaphoreType.DMA,
            pltpu.SemaphoreType.DMA,
            pltpu.SemaphoreType.DMA,
            pltpu.SemaphoreType.DMA,
            pltpu.SemaphoreType.DMA,
            pltpu.SemaphoreType.DMA,
        ],
    )
    def edge_kernel(h_hbm, si_hbm, di_hbm, w_hbm, dist_hbm, cv_hbm,
                    sidx, didx, sra, dra, srb, drb, src_, drc, distb, cvb,
                    pv, wv, ctab, sa0, sa1, sb0, sb1, sc0, sc1):
        wid = lax.axis_index("s") * NC + lax.axis_index("c")
        base = wid * EPT

        pltpu.sync_copy(si_hbm.at[pl.ds(base, EPT)], sidx)
        pltpu.sync_copy(di_hbm.at[pl.ds(base, EPT)], didx)

        pltpu.sync_copy(w_hbm, wv)
        ew = jnp.exp(wv[...])
        wv[...] = ew
        s = jnp.zeros((L,), jnp.float32)
        for h in range(H):
            s = s + plsc.load_gather(wv, [jnp.full((L,), h + 1, jnp.int32)])
        pv[...] = ew / s

        iota16 = lax.iota(jnp.int32, L)

        def fill_body(t, carry):
            hp = t // D
            d = t - hp * D
            rot = jnp.bitwise_and(iota16 + d, 15) + (hp * D)
            ctab[pl.ds(t * L, L)] = rot
            return carry

        lax.fori_loop(0, HP * D, fill_body, 0)

        def descs(k, sr, dr, s0, s1):
            cs = pltpu.make_async_copy(h_hbm.at[sidx.at[pl.ds(k * C, C)]],
                                       sr, s0)
            cd = pltpu.make_async_copy(h_hbm.at[didx.at[pl.ds(k * C, C)]],
                                       dr, s1)
            return cs, cd

        def issue(k, sr, dr, s0, s1):
            cs, cd = descs(k, sr, dr, s0, s1)
            cs.start()
            cd.start()

        def compute(k, sr, dr):
            co = k * C

            def group_body(g, carry):
                rowi = iota16 + g * L
                wsum = jnp.zeros((L,), jnp.float32)
                s1 = jnp.zeros((L,), jnp.float32)
                s2 = jnp.zeros((L,), jnp.float32)
                for hp in range(HP):
                    alo = jnp.zeros((L,), jnp.float32)
                    ahi = jnp.zeros((L,), jnp.float32)
                    for d in range(D):
                        coli = ctab[pl.ds((hp * D + d) * L, L)]
                        sv = plsc.load_gather(sr, [rowi, coli])
                        dv = plsc.load_gather(dr, [rowi, coli])
                        df2 = (plsc.bitcast(dv, jnp.bfloat16)
                               - plsc.bitcast(sv, jnp.bfloat16))
                        sq2 = df2 * df2
                        sql, sqh = plsc.unpack(
                            sq2, format=plsc.PackFormat.INTERLEAVED)
                        alo = alo + sql
                        ahi = ahi + sqh
                    for a, h in ((alo, hp), (ahi, hp + 4)):
                        n = _sqrt16(a)
                        ph = plsc.load_gather(
                            pv, [jnp.full((L,), h + 1, jnp.int32)])
                        wsum = wsum + n * ph
                        s1 = s1 + n
                        s2 = s2 + a
                mean = s1 * (1.0 / H)
                var = s2 * (1.0 / H) - mean * mean
                std = _sqrt16(var)
                distb[pl.ds(co + g * L, L)] = wsum
                cvb[pl.ds(co + g * L, L)] = std / (mean + 1.0)
                return carry

            lax.fori_loop(0, GPC, group_body, 0)

        sets = ((sra, dra, sa0, sa1), (srb, drb, sb0, sb1),
                (src_, drc, sc0, sc1))

        def wait_compute(k, st):
            cc = descs(k, *st)
            cc[0].wait()
            cc[1].wait()
            compute(k, st[0], st[1])

        issue(0, *sets[0])
        issue(1, *sets[1])
        issue(2, *sets[2])
        wait_compute(0, sets[0])
        issue(3, *sets[0])

        def tri_body(i, carry):
            for off in range(3):
                k = i * 3 + 1 + off
                st = sets[(1 + off) % 3]
                wait_compute(k, st)

                @pl.when(k + 3 < NCH)
                def _():
                    issue(k + 3, *st)
            return carry

        lax.fori_loop(0, NTRI, tri_body, 0)
        wait_compute(NCH - 1, sets[(NCH - 1) % 3])

        pltpu.sync_copy(distb, dist_hbm.at[pl.ds(base, EPT)])
        pltpu.sync_copy(cvb, cv_hbm.at[pl.ds(base, EPT)])

    return edge_kernel


_edge_kernel = _make_kernel()


@jax.jit
def kernel(h, edge_index, w):
    hb = h.astype(jnp.bfloat16)
    hpair = jnp.stack([hb[:, :HP, :], hb[:, HP:, :]], axis=-1)
    h2 = lax.bitcast_convert_type(hpair, jnp.int32).reshape(N, CW)
    si = edge_index[0]
    di = edge_index[1]
    wpad = jnp.pad(w, (1, L - H - 1), constant_values=-1e30)
    dist, cv = _edge_kernel(h2, si, di, wpad)
    return dist[:, None], cv


# speedup vs baseline: 30.2037x; 1.0003x over previous
import functools

import jax
import jax.numpy as jnp
from jax import lax
from jax.experimental import pallas as pl
from jax.experimental.pallas import tpu as pltpu
from jax.experimental.pallas import tpu_sc as plsc

N = 10000
E = 320000
H = 8
D = 16
ROW = H * D
CW = ROW // 2
HP = H // 2
L = 16
NC = 2
NS = 16
NW = NC * NS
EPT = E // NW
C = 80
NCH = EPT // C
GPC = C // L
NTRI = (NCH - 2) // 3


def _sqrt16(x, iters=2):
    i = lax.bitcast_convert_type(x, jnp.int32)
    y = lax.bitcast_convert_type(
        jnp.int32(0x5F3759DF) - lax.shift_right_arithmetic(i, 1), jnp.float32)
    for _ in range(iters):
        y = y * (1.5 - 0.5 * x * y * y)
    return jnp.where(x > 0.0, x * y, 0.0)


def _make_kernel():
    mesh = plsc.VectorSubcoreMesh(
        core_axis_name="c", subcore_axis_name="s", num_cores=NC,
        num_subcores=NS)

    @functools.partial(
        pl.kernel,
        out_type=(
            jax.ShapeDtypeStruct((E,), jnp.float32),
            jax.ShapeDtypeStruct((E,), jnp.float32),
        ),
        mesh=mesh,
        compiler_params=pltpu.CompilerParams(needs_layout_passes=False,
                                             use_tc_tiling_on_sc=False),
        scratch_types=[
            pltpu.VMEM((EPT,), jnp.int32),
            pltpu.VMEM((EPT,), jnp.int32),
            pltpu.VMEM((C, CW), jnp.int32),
            pltpu.VMEM((C, CW), jnp.int32),
            pltpu.VMEM((C, CW), jnp.int32),
            pltpu.VMEM((C, CW), jnp.int32),
            pltpu.VMEM((C, CW), jnp.int32),
            pltpu.VMEM((C, CW), jnp.int32),
            pltpu.VMEM((EPT,), jnp.float32),
            pltpu.VMEM((EPT,), jnp.float32),
            pltpu.VMEM((L,), jnp.float32),
            pltpu.VMEM((L,), jnp.float32),
            pltpu.VMEM((HP * D * L,), jnp.int32),
            pltpu.SemaphoreType.DMA,
            pltpu.SemaphoreType.DMA,
            pltpu.SemaphoreType.DMA,
            pltpu.SemaphoreType.DMA,
            pltpu.SemaphoreType.DMA,
            pltpu.SemaphoreType.DMA,
        ],
    )
    def edge_kernel(h_hbm, si_hbm, di_hbm, w_hbm, dist_hbm, cv_hbm,
                    sidx, didx, sra, dra, srb, drb, src_, drc, distb, cvb,
                    pv, wv, ctab, sa0, sa1, sb0, sb1, sc0, sc1):
        wid = lax.axis_index("s") * NC + lax.axis_index("c")
        base = wid * EPT

        pltpu.sync_copy(si_hbm.at[pl.ds(base, EPT)], sidx)
        pltpu.sync_copy(di_hbm.at[pl.ds(base, EPT)], didx)

        pltpu.sync_copy(w_hbm, wv)
        ew = jnp.exp(wv[...])
        wv[...] = ew
        s = jnp.zeros((L,), jnp.float32)
        for h in range(H):
            s = s + plsc.load_gather(wv, [jnp.full((L,), h + 1, jnp.int32)])
        pv[...] = ew / s

        iota16 = lax.iota(jnp.int32, L)

        def fill_body(t, carry):
            hp = t // D
            d = t - hp * D
            rot = jnp.bitwise_and(iota16 + d, 15) + (hp * D)
            ctab[pl.ds(t * L, L)] = rot
            return carry

        lax.fori_loop(0, HP * D, fill_body, 0)

        def descs(k, sr, dr, s0, s1):
            cs = pltpu.make_async_copy(h_hbm.at[sidx.at[pl.ds(k * C, C)]],
                                       sr, s0)
            cd = pltpu.make_async_copy(h_hbm.at[didx.at[pl.ds(k * C, C)]],
                                       dr, s1)
            return cs, cd

        def issue(k, sr, dr, s0, s1):
            cs, cd = descs(k, sr, dr, s0, s1)
            cs.start()
            cd.start()

        def compute(k, sr, dr):
            co = k * C

            def group_body(g, carry):
                rowi = iota16 + g * L
                wsum = jnp.zeros((L,), jnp.float32)
                s1 = jnp.zeros((L,), jnp.float32)
                s2 = jnp.zeros((L,), jnp.float32)
                for hp in range(HP):
                    alo = jnp.zeros((L,), jnp.float32)
                    ahi = jnp.zeros((L,), jnp.float32)
                    for d in range(D):
                        coli = ctab[pl.ds((hp * D + d) * L, L)]
                        sv = plsc.load_gather(sr, [rowi, coli])
                        dv = plsc.load_gather(dr, [rowi, coli])
                        df2 = (plsc.bitcast(dv, jnp.bfloat16)
                               - plsc.bitcast(sv, jnp.bfloat16))
                        sq2 = df2 * df2
                        sql, sqh = plsc.unpack(
                            sq2, format=plsc.PackFormat.INTERLEAVED)
                        alo = alo + sql
                        ahi = ahi + sqh
                    for a, h in ((alo, hp), (ahi, hp + 4)):
                        n = _sqrt16(a)
                        ph = plsc.load_gather(
                            pv, [jnp.full((L,), h + 1, jnp.int32)])
                        wsum = wsum + n * ph
                        s1 = s1 + n
                        s2 = s2 + a
                mean = s1 * (1.0 / H)
                var = s2 * (1.0 / H) - mean * mean
                std = _sqrt16(var)
                distb[pl.ds(co + g * L, L)] = wsum
                cvb[pl.ds(co + g * L, L)] = std / (mean + 1.0)
                return carry

            lax.fori_loop(0, GPC, group_body, 0)

        sets = ((sra, dra, sa0, sa1), (srb, drb, sb0, sb1),
                (src_, drc, sc0, sc1))

        def wait_compute(k, st):
            cc = descs(k, *st)
            cc[0].wait()
            cc[1].wait()
            compute(k, st[0], st[1])

        issue(0, *sets[0])
        issue(1, *sets[1])
        issue(2, *sets[2])
        wait_compute(0, sets[0])
        issue(3, *sets[0])

        def tri_body(i, carry):
            for off in range(3):
                k = i * 3 + 1 + off
                st = sets[(1 + off) % 3]
                wait_compute(k, st)

                @pl.when(k + 3 < NCH)
                def _():
                    issue(k + 3, *st)
            return carry

        lax.fori_loop(0, NTRI, tri_body, 0)
        wait_compute(NCH - 1, sets[(NCH - 1) % 3])

        pltpu.sync_copy(distb, dist_hbm.at[pl.ds(base, EPT)])
        pltpu.sync_copy(cvb, cv_hbm.at[pl.ds(base, EPT)])

    return edge_kernel


_edge_kernel = _make_kernel()


@jax.jit
def kernel(h, edge_index, w):
    hb = h.astype(jnp.bfloat16)
    hpair = jnp.stack([hb[:, :HP, :], hb[:, HP:, :]], axis=-1)
    h2 = lax.bitcast_convert_type(hpair, jnp.int32).reshape(N, CW)
    si = edge_index[0]
    di = edge_index[1]
    wpad = jnp.pad(w, (1, L - H - 1), constant_values=-1e30)
    dist, cv = _edge_kernel(h2, si, di, wpad)
    return dist[:, None], cv
